# baseline (speedup 1.0000x reference)
import functools

import jax
import jax.numpy as jnp
from jax import lax
from jax.experimental import pallas as pl
from jax.experimental.pallas import tpu as pltpu
from jax.experimental.pallas import tpu_sc as plsc

_G = 128
_NB = 200
_R = 4


@functools.cache
def _build(n_idx_rows, d, n_table_rows):
    mesh = plsc.VectorSubcoreMesh(core_axis_name="c", subcore_axis_name="s")
    n_workers = 32
    rows_per_w = n_idx_rows // n_workers
    n_blocks = rows_per_w // _NB

    @functools.partial(
        pl.kernel,
        out_type=jax.ShapeDtypeStruct((n_idx_rows * _G, d), jnp.float32),
        mesh=mesh,
        scratch_types=[
            pltpu.VMEM((_NB, _G), jnp.int32),
            pltpu.VMEM((_R, _G, d), jnp.float32),
            pltpu.SemaphoreType.DMA,
            pltpu.SemaphoreType.DMA((_R,)),
            pltpu.SemaphoreType.DMA((_R,)),
        ],
    )
    def body(table_hbm, idx_hbm, out_hbm, idx_v, rows_v, isem, gsem, osem):
        wid = lax.axis_index("s") * 2 + lax.axis_index("c")
        wrow = wid * rows_per_w

        n_groups = rows_per_w // _R

        def out_drain(r):
            pltpu.make_async_copy(
                rows_v.at[r], out_hbm.at[pl.ds(0, _G)], osem.at[r]).wait()

        def do_group(g, _):
            blk = g // (_NB // _R)
            jb = (g % (_NB // _R)) * _R

            @pl.when(jb == 0)
            def _stage_idx():
                pltpu.make_async_copy(
                    idx_hbm.at[pl.ds(wrow + blk * _NB, _NB)],
                    idx_v, isem).start()
                pltpu.make_async_copy(
                    idx_hbm.at[pl.ds(wrow + blk * _NB, _NB)],
                    idx_v, isem).wait()

            @pl.when(g > 0)
            def _drain_prev():
                for r in range(_R):
                    out_drain(r)

            for r in range(_R):
                pltpu.make_async_copy(
                    table_hbm.at[idx_v.at[jb + r]],
                    rows_v.at[r], gsem.at[r]).start()
            for r in range(_R):
                pltpu.make_async_copy(
                    table_hbm.at[idx_v.at[jb + r]],
                    rows_v.at[r], gsem.at[r]).wait()
                pltpu.make_async_copy(
                    rows_v.at[r],
                    out_hbm.at[pl.ds((wrow + blk * _NB + jb + r) * _G, _G)],
                    osem.at[r]).start()
            return ()

        lax.fori_loop(0, n_groups, do_group, (), unroll=False)
        for r in range(_R):
            out_drain(r)

    return body


def kernel(idx, id_table):
    b, k = idx.shape
    n_table_rows, d = id_table.shape
    flat = idx.reshape(-1).astype(jnp.int32)
    n = flat.shape[0]
    idx2d = flat.reshape(n // _G, _G)
    out = _build(n // _G, d, n_table_rows)(id_table, idx2d)
    return out.reshape(b, k, d)


# speedup vs baseline: 1.0274x; 1.0274x over previous
import functools

import jax
import jax.numpy as jnp
from jax import lax
from jax.experimental import pallas as pl
from jax.experimental.pallas import tpu as pltpu
from jax.experimental.pallas import tpu_sc as plsc

_G = 128
_NB = 200
_R = 4


@functools.cache
def _build(n_idx_rows, d, n_table_rows):
    mesh = plsc.VectorSubcoreMesh(core_axis_name="c", subcore_axis_name="s")
    n_workers = 32
    rows_per_w = n_idx_rows // n_workers
    n_blocks = rows_per_w // _NB

    @functools.partial(
        pl.kernel,
        out_type=jax.ShapeDtypeStruct((n_idx_rows * _G, d), jnp.float32),
        mesh=mesh,
        scratch_types=[
            pltpu.VMEM((_NB, _G), jnp.int32),
            pltpu.VMEM((_R, _G, d), jnp.float32),
            pltpu.SemaphoreType.DMA,
            pltpu.SemaphoreType.DMA,
            pltpu.SemaphoreType.DMA,
        ],
    )
    def body(table_hbm, idx_hbm, out_hbm, idx_v, rows_v, isem, gsem, osem):
        wid = lax.axis_index("s") * 2 + lax.axis_index("c")
        wrow = wid * rows_per_w

        def do_block(ib, _):
            row0 = wrow + ib * _NB
            cp = pltpu.make_async_copy(
                idx_hbm.at[pl.ds(row0, _NB)], idx_v, isem)
            cp.start()
            cp.wait()

            def do_group(g, _):
                j0 = g * _R
                for r in range(_R):
                    pltpu.make_async_copy(
                        table_hbm.at[idx_v.at[j0 + r]],
                        rows_v.at[r], gsem).start()
                for r in range(_R):
                    pltpu.make_async_copy(
                        table_hbm.at[idx_v.at[j0 + r]],
                        rows_v.at[r], gsem).wait()
                    pltpu.make_async_copy(
                        rows_v.at[r],
                        out_hbm.at[pl.ds((row0 + j0 + r) * _G, _G)],
                        osem).start()
                for r in range(_R):
                    pltpu.make_async_copy(
                        rows_v.at[r],
                        out_hbm.at[pl.ds((row0 + j0 + r) * _G, _G)],
                        osem).wait()
                return ()

            lax.fori_loop(0, _NB // _R, do_group, (), unroll=False)
            return ()

        lax.fori_loop(0, n_blocks, do_block, (), unroll=False)

    return body


def kernel(idx, id_table):
    b, k = idx.shape
    n_table_rows, d = id_table.shape
    flat = idx.reshape(-1).astype(jnp.int32)
    n = flat.shape[0]
    idx2d = flat.reshape(n // _G, _G)
    out = _build(n // _G, d, n_table_rows)(id_table, idx2d)
    return out.reshape(b, k, d)


# speedup vs baseline: 1.0305x; 1.0030x over previous
import functools

import jax
import jax.numpy as jnp
from jax import lax
from jax.experimental import pallas as pl
from jax.experimental.pallas import tpu as pltpu
from jax.experimental.pallas import tpu_sc as plsc

_G = 128
_NB = 200
_R = 4


@functools.cache
def _build(n_idx_rows, d, n_table_rows):
    mesh = plsc.VectorSubcoreMesh(core_axis_name="c", subcore_axis_name="s")
    n_workers = 32
    rows_per_w = n_idx_rows // n_workers
    n_blocks = rows_per_w // _NB

    @functools.partial(
        pl.kernel,
        out_type=jax.ShapeDtypeStruct((n_idx_rows * _G, d), jnp.float32),
        mesh=mesh,
        scratch_types=[
            pltpu.VMEM((_NB, _G), jnp.int32),
            pltpu.VMEM((_R, _G, d), jnp.float32),
            pltpu.SemaphoreType.DMA,
            pltpu.SemaphoreType.DMA,
            pltpu.SemaphoreType.DMA,
        ],
    )
    def body(table_hbm, idx_hbm, out_hbm, idx_v, rows_v, isem, gsem, osem):
        wid = lax.axis_index("s") * 2 + lax.axis_index("c")
        wrow = wid * rows_per_w

        n_groups = rows_per_w // _R

        def out_drain():
            pltpu.make_async_copy(
                rows_v.at[0], out_hbm.at[pl.ds(0, _G)], osem).wait()

        def do_group(g, _):
            blk = g // (_NB // _R)
            jb = (g % (_NB // _R)) * _R

            @pl.when(jb == 0)
            def _stage_idx():
                pltpu.make_async_copy(
                    idx_hbm.at[pl.ds(wrow + blk * _NB, _NB)],
                    idx_v, isem).start()
                pltpu.make_async_copy(
                    idx_hbm.at[pl.ds(wrow + blk * _NB, _NB)],
                    idx_v, isem).wait()

            @pl.when(g > 0)
            def _drain_prev():
                for _ in range(_R):
                    out_drain()

            for r in range(_R):
                pltpu.make_async_copy(
                    table_hbm.at[idx_v.at[jb + r]],
                    rows_v.at[r], gsem).start()
            for r in range(_R):
                pltpu.make_async_copy(
                    table_hbm.at[idx_v.at[jb + r]],
                    rows_v.at[r], gsem).wait()
                pltpu.make_async_copy(
                    rows_v.at[r],
                    out_hbm.at[pl.ds((wrow + blk * _NB + jb + r) * _G, _G)],
                    osem).start()
            return ()

        lax.fori_loop(0, n_groups, do_group, (), unroll=False)
        for _ in range(_R):
            out_drain()

    return body


def kernel(idx, id_table):
    b, k = idx.shape
    n_table_rows, d = id_table.shape
    flat = idx.reshape(-1).astype(jnp.int32)
    n = flat.shape[0]
    idx2d = flat.reshape(n // _G, _G)
    out = _build(n // _G, d, n_table_rows)(id_table, idx2d)
    return out.reshape(b, k, d)


# speedup vs baseline: 1.0337x; 1.0031x over previous
import functools

import jax
import jax.numpy as jnp
from jax import lax
from jax.experimental import pallas as pl
from jax.experimental.pallas import tpu as pltpu
from jax.experimental.pallas import tpu_sc as plsc

_G = 128
_NB = 200
_R = 5


@functools.cache
def _build(n_idx_rows, d, n_table_rows):
    mesh = plsc.VectorSubcoreMesh(core_axis_name="c", subcore_axis_name="s")
    n_workers = 32
    rows_per_w = n_idx_rows // n_workers
    n_blocks = rows_per_w // _NB

    @functools.partial(
        pl.kernel,
        out_type=jax.ShapeDtypeStruct((n_idx_rows * _G, d), jnp.float32),
        mesh=mesh,
        scratch_types=[
            pltpu.VMEM((_NB, _G), jnp.int32),
            pltpu.VMEM((_R, _G, d), jnp.float32),
            pltpu.SemaphoreType.DMA,
            pltpu.SemaphoreType.DMA,
            pltpu.SemaphoreType.DMA,
        ],
    )
    def body(table_hbm, idx_hbm, out_hbm, idx_v, rows_v, isem, gsem, osem):
        wid = lax.axis_index("s") * 2 + lax.axis_index("c")
        wrow = wid * rows_per_w

        n_groups = rows_per_w // _R

        def out_drain():
            pltpu.make_async_copy(
                rows_v.at[0], out_hbm.at[pl.ds(0, _G)], osem).wait()

        def do_group(g, _):
            blk = g // (_NB // _R)
            jb = (g % (_NB // _R)) * _R

            @pl.when(jb == 0)
            def _stage_idx():
                pltpu.make_async_copy(
                    idx_hbm.at[pl.ds(wrow + blk * _NB, _NB)],
                    idx_v, isem).start()
                pltpu.make_async_copy(
                    idx_hbm.at[pl.ds(wrow + blk * _NB, _NB)],
                    idx_v, isem).wait()

            @pl.when(g > 0)
            def _drain_prev():
                for _ in range(_R):
                    out_drain()

            for r in range(_R):
                pltpu.make_async_copy(
                    table_hbm.at[idx_v.at[jb + r]],
                    rows_v.at[r], gsem).start()
            for r in range(_R):
                pltpu.make_async_copy(
                    table_hbm.at[idx_v.at[jb + r]],
                    rows_v.at[r], gsem).wait()
                pltpu.make_async_copy(
                    rows_v.at[r],
                    out_hbm.at[pl.ds((wrow + blk * _NB + jb + r) * _G, _G)],
                    osem).start()
            return ()

        lax.fori_loop(0, n_groups, do_group, (), unroll=False)
        for _ in range(_R):
            out_drain()

    return body


def kernel(idx, id_table):
    b, k = idx.shape
    n_table_rows, d = id_table.shape
    flat = idx.reshape(-1).astype(jnp.int32)
    n = flat.shape[0]
    idx2d = flat.reshape(n // _G, _G)
    out = _build(n // _G, d, n_table_rows)(id_table, idx2d)
    return out.reshape(b, k, d)


# speedup vs baseline: 1.0605x; 1.0259x over previous
import functools

import jax
import jax.numpy as jnp
from jax import lax
from jax.experimental import pallas as pl
from jax.experimental.pallas import tpu as pltpu
from jax.experimental.pallas import tpu_sc as plsc

_GA = 128
_SB = 64
_R = 4


@functools.cache
def _build(b, k, d, n_table_rows):
    mesh = plsc.VectorSubcoreMesh(core_axis_name="c", subcore_axis_name="s")
    n_workers = 32
    gb = k - _GA
    rows_per_w = b // n_workers
    n_blocks = rows_per_w // _SB

    @functools.partial(
        pl.kernel,
        out_type=jax.ShapeDtypeStruct((b, k, d), jnp.float32),
        mesh=mesh,
        scratch_types=[
            pltpu.VMEM((_SB, k), jnp.int32),
            pltpu.VMEM((_R, _GA, d), jnp.float32),
            pltpu.VMEM((_R, gb, d), jnp.float32),
            pltpu.SemaphoreType.DMA,
            pltpu.SemaphoreType.DMA,
            pltpu.SemaphoreType.DMA,
        ],
    )
    def body(table_hbm, idx_hbm, out_hbm, idx_v, ra_v, rb_v,
             isem, gsem, osem):
        wid = lax.axis_index("s") * 2 + lax.axis_index("c")
        wrow = wid * rows_per_w

        def do_block(ib, _):
            row0 = wrow + ib * _SB
            cp = pltpu.make_async_copy(
                idx_hbm.at[pl.ds(row0, _SB)], idx_v, isem)
            cp.start()
            cp.wait()

            def do_group(g, _):
                j0 = g * _R
                for r in range(_R):
                    pltpu.make_async_copy(
                        table_hbm.at[idx_v.at[j0 + r, pl.ds(0, _GA)]],
                        ra_v.at[r], gsem).start()
                    pltpu.make_async_copy(
                        table_hbm.at[idx_v.at[j0 + r, pl.ds(_GA, gb)]],
                        rb_v.at[r], gsem).start()
                for r in range(_R):
                    pltpu.make_async_copy(
                        table_hbm.at[idx_v.at[j0 + r, pl.ds(0, _GA)]],
                        ra_v.at[r], gsem).wait()
                    pltpu.make_async_copy(
                        ra_v.at[r],
                        out_hbm.at[row0 + j0 + r, pl.ds(0, _GA)],
                        osem).start()
                    pltpu.make_async_copy(
                        table_hbm.at[idx_v.at[j0 + r, pl.ds(_GA, gb)]],
                        rb_v.at[r], gsem).wait()
                    pltpu.make_async_copy(
                        rb_v.at[r],
                        out_hbm.at[row0 + j0 + r, pl.ds(_GA, gb)],
                        osem).start()
                for r in range(_R):
                    pltpu.make_async_copy(
                        ra_v.at[r],
                        out_hbm.at[row0 + j0 + r, pl.ds(0, _GA)],
                        osem).wait()
                    pltpu.make_async_copy(
                        rb_v.at[r],
                        out_hbm.at[row0 + j0 + r, pl.ds(_GA, gb)],
                        osem).wait()
                return ()

            lax.fori_loop(0, _SB // _R, do_group, (), unroll=False)
            return ()

        lax.fori_loop(0, n_blocks, do_block, (), unroll=False)

    return body


def kernel(idx, id_table):
    b, k = idx.shape
    n_table_rows, d = id_table.shape
    return _build(b, k, d, n_table_rows)(id_table, idx.astype(jnp.int32))
